# baseline (speedup 1.0000x reference)
import functools

import jax
import jax.numpy as jnp
from jax import lax
from jax.experimental import pallas as pl
from jax.experimental.pallas import tpu as pltpu
from jax.experimental.pallas import tpu_sc as plsc

F = 26
D = 16
V = 100000
KR = F * D
NC = 4
NI = F * NC
PAIRS = [(i, j) for i in range(F) for j in range(i + 1, F)]
NPAIR = len(PAIRS)
NCOL = (NPAIR + F) * D

NB = 128
NFULL = V // NB
VTAIL = V - NFULL * NB
UNITS = NC * NFULL
OP = NB + 1


def _chunk(s, t):
    return s * NC + t // 8, (t % 8) * D


@functools.lru_cache(maxsize=None)
def _transpose_call():
    info = plsc.get_sparse_core_info()
    nw = info.num_cores * info.num_subcores
    mesh = plsc.VectorSubcoreMesh(core_axis_name="c", subcore_axis_name="s")

    @functools.partial(
        pl.kernel,
        out_type=jax.ShapeDtypeStruct((NC * V, 128), jnp.float32),
        mesh=mesh,
        scratch_types=[
            pltpu.VMEM((2, NB, NB), jnp.float32),
            pltpu.VMEM((2, NB, OP), jnp.float32),
            pltpu.SemaphoreType.DMA,
            pltpu.SemaphoreType.DMA,
            pltpu.SemaphoreType.DMA,
            pltpu.SemaphoreType.DMA,
        ],
        compiler_params=pltpu.CompilerParams(
            use_tc_tiling_on_sc=True, needs_layout_passes=False),
    )
    def k(tv_hbm, tail_hbm, out_hbm, inb, outb, isem0, isem1, osem0, osem1):
        wid = lax.axis_index("s") * info.num_cores + lax.axis_index("c")
        isems = (isem0, isem1)
        osems = (osem0, osem1)
        iota = lax.iota(jnp.int32, 16)

        def unit(u):
            g = u * nw + wid
            return g // NFULL, g % NFULL

        def in_copy(u, p, sem):
            c, vb = unit(u)
            col = vb * NB

            @pl.when(c == NC - 1)
            def _():
                pltpu.make_async_copy(
                    tv_hbm.at[pl.ds(384, 32), pl.ds(col, NB)],
                    inb.at[p].at[pl.ds(0, 32)], sem).start()

            @pl.when(c < NC - 1)
            def _():
                pltpu.make_async_copy(
                    tv_hbm.at[pl.ds(c * NB, NB), pl.ds(col, NB)],
                    inb.at[p], sem).start()

        def in_wait(u, p, sem):
            c, vb = unit(u)
            col = vb * NB

            @pl.when(c == NC - 1)
            def _():
                pltpu.make_async_copy(
                    tv_hbm.at[pl.ds(384, 32), pl.ds(col, NB)],
                    inb.at[p].at[pl.ds(0, 32)], sem).wait()

            @pl.when(c < NC - 1)
            def _():
                pltpu.make_async_copy(
                    tv_hbm.at[pl.ds(c * NB, NB), pl.ds(col, NB)],
                    inb.at[p], sem).wait()

        def out_desc(u, p, sem):
            c, vb = unit(u)
            return pltpu.make_async_copy(
                outb.at[p].at[:, pl.ds(0, NB)],
                out_hbm.at[pl.ds(c * V + vb * NB, NB)], sem)

        rowidx = [iota + vg * 16 for vg in range(8)]

        def transpose_block(p, nv):
            src = inb.at[p]
            dst = outb.at[p]

            def kbody(k, carry):
                colk = jnp.full((16,), k, jnp.int32)
                vals = [src[k, pl.ds(vg * 16, 16)] for vg in range(8)]
                for vg in range(8):
                    plsc.store_scatter(dst, [rowidx[vg], colk], vals[vg])
                return carry

            lax.fori_loop(0, NB, kbody, 0)

        in_copy(0, 0, isems[0])

        def body(k2, carry):
            for p in range(2):
                u = k2 * 2 + p

                @pl.when(u * nw + wid < UNITS)
                def _():
                    nxt = u + 1

                    @pl.when(nxt * nw + wid < UNITS)
                    def _():
                        in_copy(nxt, (p + 1) % 2, isems[(p + 1) % 2])

                    in_wait(u, p, isems[p])

                    @pl.when(u >= 2)
                    def _():
                        out_desc(u - 2, p, osems[p]).wait()

                    transpose_block(p, NB)
                    out_desc(u, p, osems[p]).start()
            return carry

        nloop = (UNITS + nw - 1) // nw
        lax.fori_loop(0, nloop // 2, body, 0)

        for p in range(2):
            last = nloop - 2 + p

            @pl.when(last * nw + wid < UNITS)
            def _():
                out_desc(last, p, osems[p]).wait()

            @pl.when((last * nw + wid >= UNITS)
                     & ((last - 2) * nw + wid < UNITS))
            def _():
                out_desc(last - 2, p, osems[p]).wait()

        for cc in range(NC):
            @pl.when(wid == 28 + cc)
            def _():
                pltpu.sync_copy(tail_hbm.at[cc],
                                inb.at[0].at[pl.ds(0, VTAIL)])
                pltpu.sync_copy(
                    inb.at[0].at[pl.ds(0, VTAIL)],
                    out_hbm.at[pl.ds(cc * V + NFULL * NB, VTAIL)])

    return k


@functools.lru_cache(maxsize=None)
def _gather_call(batch):
    info = plsc.get_sparse_core_info()
    nw = info.num_cores * info.num_subcores
    assert batch % nw == 0
    per_w = batch // nw
    mesh = plsc.VectorSubcoreMesh(core_axis_name="c", subcore_axis_name="s")

    @functools.partial(
        pl.kernel,
        out_type=jax.ShapeDtypeStruct((batch, NCOL), jnp.float32),
        mesh=mesh,
        scratch_types=[
            pltpu.VMEM((per_w, NI), jnp.int32),
            pltpu.VMEM((4, NI, 128), jnp.float32),
            pltpu.VMEM((2, NCOL), jnp.float32),
            pltpu.SemaphoreType.DMA,
            pltpu.SemaphoreType.DMA,
            pltpu.SemaphoreType.DMA,
            pltpu.SemaphoreType.DMA,
            pltpu.SemaphoreType.DMA,
            pltpu.SemaphoreType.DMA,
        ],
        compiler_params=pltpu.CompilerParams(use_tc_tiling_on_sc=True),
    )
    def k(tabt_hbm, xq_hbm, out_hbm, xv, rows_v, outb_v,
          gsem0, gsem1, gsem2, gsem3, osem0, osem1):
        wid = lax.axis_index("s") * info.num_cores + lax.axis_index("c")
        base = wid * per_w
        gsems = (gsem0, gsem1, gsem2, gsem3)
        osems = (osem0, osem1)
        pltpu.sync_copy(xq_hbm.at[pl.ds(base, per_w)], xv)

        def gather_desc(e, p, sem):
            return pltpu.make_async_copy(
                tabt_hbm.at[xv.at[e]], rows_v.at[p], sem)

        def compute(q):
            rows = rows_v.at[q]
            outb = outb_v.at[q % 2]
            for q, (i, j) in enumerate(PAIRS):
                ra, ca = _chunk(i, j)
                rb, cb = _chunk(j, i)
                outb[pl.ds(q * D, D)] = (
                    rows[ra, pl.ds(ca, D)] * rows[rb, pl.ds(cb, D)]
                )
            for i in range(F):
                r, c = _chunk(i, i)
                outb[pl.ds((NPAIR + i) * D, D)] = rows[r, pl.ds(c, D)]

        gather_desc(0, 0, gsems[0]).start()
        gather_desc(1, 1, gsems[1]).start()

        def body(k4, carry):
            for q in range(4):
                e = k4 * 4 + q
                p = q % 2

                @pl.when(e + 2 < per_w)
                def _():
                    gather_desc(e + 2, (q + 2) % 4, gsems[(q + 2) % 4]).start()

                gather_desc(e, q, gsems[q]).wait()

                @pl.when(e >= 2)
                def _():
                    pltpu.make_async_copy(
                        outb_v.at[p], out_hbm.at[base + e - 2], osems[p]).wait()

                compute(q)
                pltpu.make_async_copy(
                    outb_v.at[p], out_hbm.at[base + e], osems[p]).start()
            return carry

        lax.fori_loop(0, per_w // 4, body, 0)
        for p in range(2):
            pltpu.make_async_copy(
                outb_v.at[p], out_hbm.at[base + per_w - 2 + p], osems[p]).wait()

    return k


def kernel(x, tables):
    batch = x.shape[0]
    tv = tables.transpose(0, 2, 1).reshape(KR, V)
    tvp = jnp.concatenate([tv, jnp.zeros((NC * 128 - KR, V), jnp.float32)], 0)
    tabt = lax.transpose(tvp.reshape(NC, 128, V), (0, 2, 1))
    tabt = tabt.reshape(NC * V, 128)
    xq = (x.astype(jnp.int32)[:, :, None]
          + (jnp.arange(NC, dtype=jnp.int32) * V)[None, None, :]
          ).reshape(batch, NI)
    return _gather_call(batch)(tabt, xq)


# speedup vs baseline: 1.0200x; 1.0200x over previous
import functools

import jax
import jax.numpy as jnp
from jax import lax
from jax.experimental import pallas as pl
from jax.experimental.pallas import tpu as pltpu
from jax.experimental.pallas import tpu_sc as plsc

F = 26
D = 16
V = 100000
KR = F * D
NC = 4
NI = F * NC
PAIRS = [(i, j) for i in range(F) for j in range(i + 1, F)]
NPAIR = len(PAIRS)
NCOL = (NPAIR + F) * D

NB = 128
NFULL = V // NB
VTAIL = V - NFULL * NB
UNITS = NC * NFULL
OP = NB + 1


def _chunk(s, t):
    return s * NC + t // 8, (t % 8) * D


@functools.lru_cache(maxsize=None)
def _transpose_call():
    info = plsc.get_sparse_core_info()
    nw = info.num_cores * info.num_subcores
    mesh = plsc.VectorSubcoreMesh(core_axis_name="c", subcore_axis_name="s")

    @functools.partial(
        pl.kernel,
        out_type=jax.ShapeDtypeStruct((NC * V, 128), jnp.float32),
        mesh=mesh,
        scratch_types=[
            pltpu.VMEM((2, NB, NB), jnp.float32),
            pltpu.VMEM((2, NB, OP), jnp.float32),
            pltpu.SemaphoreType.DMA,
            pltpu.SemaphoreType.DMA,
            pltpu.SemaphoreType.DMA,
            pltpu.SemaphoreType.DMA,
        ],
        compiler_params=pltpu.CompilerParams(
            use_tc_tiling_on_sc=True, needs_layout_passes=False),
    )
    def k(tv_hbm, tail_hbm, out_hbm, inb, outb, isem0, isem1, osem0, osem1):
        wid = lax.axis_index("s") * info.num_cores + lax.axis_index("c")
        isems = (isem0, isem1)
        osems = (osem0, osem1)
        iota = lax.iota(jnp.int32, 16)

        def unit(u):
            g = u * nw + wid
            return g // NFULL, g % NFULL

        def in_copy(u, p, sem):
            c, vb = unit(u)
            col = vb * NB

            @pl.when(c == NC - 1)
            def _():
                pltpu.make_async_copy(
                    tv_hbm.at[pl.ds(384, 32), pl.ds(col, NB)],
                    inb.at[p].at[pl.ds(0, 32)], sem).start()

            @pl.when(c < NC - 1)
            def _():
                pltpu.make_async_copy(
                    tv_hbm.at[pl.ds(c * NB, NB), pl.ds(col, NB)],
                    inb.at[p], sem).start()

        def in_wait(u, p, sem):
            c, vb = unit(u)
            col = vb * NB

            @pl.when(c == NC - 1)
            def _():
                pltpu.make_async_copy(
                    tv_hbm.at[pl.ds(384, 32), pl.ds(col, NB)],
                    inb.at[p].at[pl.ds(0, 32)], sem).wait()

            @pl.when(c < NC - 1)
            def _():
                pltpu.make_async_copy(
                    tv_hbm.at[pl.ds(c * NB, NB), pl.ds(col, NB)],
                    inb.at[p], sem).wait()

        def out_desc(u, p, sem):
            c, vb = unit(u)
            return pltpu.make_async_copy(
                outb.at[p].at[:, pl.ds(0, NB)],
                out_hbm.at[pl.ds(c * V + vb * NB, NB)], sem)

        rowidx = [iota + vg * 16 for vg in range(8)]

        def transpose_block(p, nv):
            src = inb.at[p]
            dst = outb.at[p]

            def kbody(k, carry):
                colk = jnp.full((16,), k, jnp.int32)
                vals = [src[k, pl.ds(vg * 16, 16)] for vg in range(8)]
                for vg in range(8):
                    plsc.store_scatter(dst, [rowidx[vg], colk], vals[vg])
                return carry

            lax.fori_loop(0, NB, kbody, 0)

        in_copy(0, 0, isems[0])

        def body(k2, carry):
            for p in range(2):
                u = k2 * 2 + p

                @pl.when(u * nw + wid < UNITS)
                def _():
                    nxt = u + 1

                    @pl.when(nxt * nw + wid < UNITS)
                    def _():
                        in_copy(nxt, (p + 1) % 2, isems[(p + 1) % 2])

                    in_wait(u, p, isems[p])

                    @pl.when(u >= 2)
                    def _():
                        out_desc(u - 2, p, osems[p]).wait()

                    transpose_block(p, NB)
                    out_desc(u, p, osems[p]).start()
            return carry

        nloop = (UNITS + nw - 1) // nw
        lax.fori_loop(0, nloop // 2, body, 0)

        for p in range(2):
            last = nloop - 2 + p

            @pl.when(last * nw + wid < UNITS)
            def _():
                out_desc(last, p, osems[p]).wait()

            @pl.when((last * nw + wid >= UNITS)
                     & ((last - 2) * nw + wid < UNITS))
            def _():
                out_desc(last - 2, p, osems[p]).wait()

        for cc in range(NC):
            @pl.when(wid == 28 + cc)
            def _():
                pltpu.sync_copy(tail_hbm.at[cc],
                                inb.at[0].at[pl.ds(0, VTAIL)])
                pltpu.sync_copy(
                    inb.at[0].at[pl.ds(0, VTAIL)],
                    out_hbm.at[pl.ds(cc * V + NFULL * NB, VTAIL)])

    return k


@functools.lru_cache(maxsize=None)
def _gather_call(batch):
    info = plsc.get_sparse_core_info()
    nw = info.num_cores * info.num_subcores
    assert batch % nw == 0
    per_w = batch // nw
    mesh = plsc.VectorSubcoreMesh(core_axis_name="c", subcore_axis_name="s")

    @functools.partial(
        pl.kernel,
        out_type=jax.ShapeDtypeStruct((batch, NCOL), jnp.float32),
        mesh=mesh,
        scratch_types=[
            pltpu.VMEM((per_w, NI), jnp.int32),
            pltpu.VMEM((2, NI, 128), jnp.float32),
            pltpu.VMEM((2, NCOL), jnp.float32),
            pltpu.SemaphoreType.DMA,
            pltpu.SemaphoreType.DMA,
            pltpu.SemaphoreType.DMA,
            pltpu.SemaphoreType.DMA,
        ],
        compiler_params=pltpu.CompilerParams(use_tc_tiling_on_sc=True),
    )
    def k(tabt_hbm, xq_hbm, out_hbm, xv, rows_v, outb_v,
          gsem0, gsem1, osem0, osem1):
        wid = lax.axis_index("s") * info.num_cores + lax.axis_index("c")
        base = wid * per_w
        gsems = (gsem0, gsem1)
        osems = (osem0, osem1)
        pltpu.sync_copy(xq_hbm.at[pl.ds(base, per_w)], xv)

        def gather_desc(e, p, sem):
            return pltpu.make_async_copy(
                tabt_hbm.at[xv.at[e]], rows_v.at[p], sem)

        def compute(p):
            rows = rows_v.at[p]
            outb = outb_v.at[p]
            for q, (i, j) in enumerate(PAIRS):
                ra, ca = _chunk(i, j)
                rb, cb = _chunk(j, i)
                outb[pl.ds(q * D, D)] = (
                    rows[ra, pl.ds(ca, D)] * rows[rb, pl.ds(cb, D)]
                )
            for i in range(F):
                r, c = _chunk(i, i)
                outb[pl.ds((NPAIR + i) * D, D)] = rows[r, pl.ds(c, D)]

        gather_desc(0, 0, gsems[0]).start()

        def body(k2, carry):
            for p in range(2):
                e = k2 * 2 + p

                @pl.when(e + 1 < per_w)
                def _():
                    gather_desc(e + 1, (p + 1) % 2, gsems[(p + 1) % 2]).start()

                gather_desc(e, p, gsems[p]).wait()

                @pl.when(e >= 2)
                def _():
                    pltpu.make_async_copy(
                        outb_v.at[p], out_hbm.at[base + e - 2], osems[p]).wait()

                compute(p)
                pltpu.make_async_copy(
                    outb_v.at[p], out_hbm.at[base + e], osems[p]).start()
            return carry

        lax.fori_loop(0, per_w // 2, body, 0)
        for p in range(2):
            pltpu.make_async_copy(
                outb_v.at[p], out_hbm.at[base + per_w - 2 + p], osems[p]).wait()

    return k


def kernel(x, tables):
    batch = x.shape[0]
    tv = tables.transpose(0, 2, 1).reshape(KR, V)
    tvp = jnp.concatenate([tv, jnp.zeros((NC * 128 - KR, V), jnp.float32)], 0)
    tabt = lax.transpose(tvp.reshape(NC, 128, V), (0, 2, 1))
    tabt = tabt.reshape(NC * V, 128)
    xq = (x.astype(jnp.int32)[:, :, None]
          + (jnp.arange(NC, dtype=jnp.int32) * V)[None, None, :]
          ).reshape(batch, NI)
    return _gather_call(batch)(tabt, xq)
